# baseline (speedup 1.0000x reference)
import math

import jax
import jax.numpy as jnp
import numpy as np
from jax import lax
from jax.experimental import pallas as pl
from jax.experimental.pallas import tpu as pltpu
from jax.experimental.pallas import tpu_sc as plsc

D_MODEL_C = 768
N_HEADS_C = 12
N_EXPERTS_C = 8
D_HEAD_C = 64
HE_C = N_HEADS_C * N_EXPERTS_C
SEQ_C = 2048

SBLK = 256
VBLK = 512
ABLK = 512
OBLK = 512
HPB = 2
DVA = 128

_F32 = jnp.float32

_SC_WORKERS = 32
_SC_TOTAL = 2 * SEQ_C * HE_C
_SC_CHUNK = _SC_TOTAL // _SC_WORKERS
_SC_NV = _SC_CHUNK // 16


def _tri_bias_np(n):
    b = np.zeros((2, n, n), dtype=np.float32)
    r = np.arange(n)
    b[0][r[:, None] < r[None, :]] = -1e30
    return b


_TRI_NP = _tri_bias_np(ABLK)


def _gate_expand_np():
    M = np.zeros((HE_C, N_EXPERTS_C * D_MODEL_C), dtype=np.float32)
    for h in range(N_HEADS_C):
        for e in range(N_EXPERTS_C):
            base = e * D_MODEL_C + h * D_HEAD_C
            M[h * 8 + e, base:base + D_HEAD_C] = 1.0
    return M


_GEXP_NP = _gate_expand_np()


_E16 = np.arange(16) % 8
_WRAP_NP = [(_E16 < 8 - d).astype(np.float32) for d in range(8)]
_TIE_NP = [(((_E16 + d) % 8) < _E16).astype(np.float32) for d in range(8)]


def _sc_route_body(sel_ref, mask_ref, ibuf, obuf, sem_in, sem_out):
    c = lax.axis_index("c")
    s = lax.axis_index("s")
    base = (c * 16 + s) * _SC_CHUNK
    cin = pltpu.make_async_copy(
        sel_ref.at[pl.ds(base, _SC_CHUNK + 16)], ibuf, sem_in)
    cin.start()
    cin.wait()
    one = jnp.ones((16,), _F32)
    zero = jnp.zeros((16,), _F32)
    two = jnp.full((16,), 2.0, _F32)
    e_id = jnp.bitwise_and(lax.broadcasted_iota(jnp.int32, (16,), 0), 7)

    def body(i, carry):
        b = 8 + i * 16
        x = ibuf[pl.ds(b, 16)]
        rank = zero
        for d in range(1, 8):
            lo = ibuf[pl.ds(b + d, 16)]
            hi = ibuf[pl.ds(b + d - 8, 16)]
            pd = jnp.where(e_id < 8 - d, lo, hi)
            pe = jnp.bitwise_and(e_id + d, 7)
            beats = (pd > x) | ((pd == x) & (pe < e_id))
            rank = rank + jnp.where(beats, one, zero)
        obuf[pl.ds(i * 16, 16)] = jnp.where(rank < two, one, zero)
        return carry

    lax.fori_loop(0, _SC_NV, body, 0)
    cout = pltpu.make_async_copy(
        obuf, mask_ref.at[pl.ds(base, _SC_CHUNK)], sem_out)
    cout.start()
    cout.wait()


_sc_route = pl.kernel(
    _sc_route_body,
    out_type=jax.ShapeDtypeStruct((_SC_TOTAL,), _F32),
    mesh=plsc.VectorSubcoreMesh(core_axis_name="c", subcore_axis_name="s"),
    scratch_types=[
        pltpu.VMEM((_SC_CHUNK + 16,), _F32),
        pltpu.VMEM((_SC_CHUNK,), _F32),
        pltpu.SemaphoreType.DMA,
        pltpu.SemaphoreType.DMA,
    ],
)


def _proj_body(x_ref, wq_ref, wk_ref, svw_ref, sow_ref,
               q_ref, k_ref, sv_ref, so_ref):
    xb = x_ref[...]
    qb = lax.dot_general(xb, wq_ref[...], (((1,), (1,)), ((), ())),
                         preferred_element_type=_F32)
    kb = lax.dot_general(xb, wk_ref[...], (((1,), (1,)), ((), ())),
                         preferred_element_type=_F32)
    sv_ref[...] = lax.dot_general(xb, svw_ref[...], (((1,), (1,)), ((), ())),
                                  preferred_element_type=_F32)
    so_ref[...] = lax.dot_general(xb, sow_ref[...], (((1,), (1,)), ((), ())),
                                  preferred_element_type=_F32)
    qscale = math.log2(math.e) / math.sqrt(D_HEAD_C)
    for h in range(N_HEADS_C):
        q_ref[h] = qb[:, h * D_HEAD_C:(h + 1) * D_HEAD_C] * qscale
        k_ref[h] = kb[:, h * D_HEAD_C:(h + 1) * D_HEAD_C]


def _v_body(x_ref, wv_ref, sv_ref, mv_ref, v_ref):
    g = jax.nn.sigmoid(sv_ref[...]) * mv_ref[...]
    onecol = (lax.broadcasted_iota(jnp.int32, (VBLK, DVA - D_HEAD_C), 1)
              == 0).astype(_F32)
    xb = x_ref[...]
    for h in range(N_HEADS_C):
        t = lax.dot_general(
            xb, wv_ref[:, h * 512:(h + 1) * 512],
            (((1,), (0,)), ((), ())), preferred_element_type=_F32)
        acc = jnp.zeros((VBLK, D_HEAD_C), dtype=_F32)
        for e in range(N_EXPERTS_C):
            gcol = g[:, h * 8 + e:h * 8 + e + 1]
            acc = acc + gcol * t[:, e * D_HEAD_C:(e + 1) * D_HEAD_C]
        v_ref[h, :, 0:D_HEAD_C] = acc
        v_ref[h, :, D_HEAD_C:DVA] = onecol


def _attn_body(q_ref, k_ref, v_ref, b_ref, o_ref, acc_sc):
    qi = pl.program_id(1)
    kj = pl.program_id(2)
    nk = pl.num_programs(2)

    @pl.when(kj <= qi)
    def _compute():
        bias = b_ref[0]
        for hh in range(HPB):
            s = lax.dot_general(q_ref[hh], k_ref[hh],
                                (((1,), (1,)), ((), ())),
                                preferred_element_type=_F32)
            p = jnp.exp2(s + bias)
            pv = lax.dot_general(p, v_ref[hh], (((1,), (0,)), ((), ())),
                                 preferred_element_type=_F32)

            @pl.when(kj == 0)
            def _first():
                acc_sc[hh] = pv

            @pl.when(kj > 0)
            def _rest():
                acc_sc[hh] = acc_sc[hh] + pv

    @pl.when(kj == nk - 1)
    def _fin():
        for hh in range(HPB):
            acc = acc_sc[hh]
            o_ref[hh] = acc[:, 0:D_HEAD_C] / acc[:, D_HEAD_C:D_HEAD_C + 1]


def _out_body(res_ref, so_ref, mo_ref, gexp_ref, wo_ref, out_ref):
    res2 = res_ref[...]
    g = jax.nn.sigmoid(so_ref[...]) * mo_ref[...]
    ge = lax.dot_general(g, gexp_ref[...], (((1,), (0,)), ((), ())),
                         preferred_element_type=_F32)
    acc = jnp.zeros((OBLK, D_MODEL_C), dtype=_F32)
    for e in range(N_EXPERTS_C):
        prod = res2 * ge[:, e * D_MODEL_C:(e + 1) * D_MODEL_C]
        acc = acc + lax.dot_general(
            prod, wo_ref[e * D_MODEL_C:(e + 1) * D_MODEL_C, :],
            (((1,), (0,)), ((), ())), preferred_element_type=_F32)
    out_ref[...] = acc


@jax.jit
def kernel(x, Wq, Wk, Wv, Wo, sel_v_w, sel_o_w):
    B, S, D = x.shape
    x2d = x.reshape(S, D)

    nsb = S // SBLK
    H, E, dh = N_HEADS_C, N_EXPERTS_C, D_HEAD_C

    q, k, sel_v, sel_o = pl.pallas_call(
        _proj_body,
        grid=(nsb,),
        in_specs=[
            pl.BlockSpec((SBLK, D), lambda i: (i, 0)),
            pl.BlockSpec((D, D), lambda i: (0, 0)),
            pl.BlockSpec((D, D), lambda i: (0, 0)),
            pl.BlockSpec((HE_C, D), lambda i: (0, 0)),
            pl.BlockSpec((HE_C, D), lambda i: (0, 0)),
        ],
        out_specs=[
            pl.BlockSpec((H, SBLK, dh), lambda i: (0, i, 0)),
            pl.BlockSpec((H, SBLK, dh), lambda i: (0, i, 0)),
            pl.BlockSpec((SBLK, HE_C), lambda i: (i, 0)),
            pl.BlockSpec((SBLK, HE_C), lambda i: (i, 0)),
        ],
        out_shape=[
            jax.ShapeDtypeStruct((H, S, dh), _F32),
            jax.ShapeDtypeStruct((H, S, dh), _F32),
            jax.ShapeDtypeStruct((S, HE_C), _F32),
            jax.ShapeDtypeStruct((S, HE_C), _F32),
        ],
    )(x2d, Wq, Wk, sel_v_w, sel_o_w)

    selpad = jnp.concatenate([
        jnp.zeros((8,), _F32), sel_v.reshape(-1), sel_o.reshape(-1),
        jnp.zeros((8,), _F32)])
    masks = _sc_route(selpad)
    mv = masks[:S * HE_C].reshape(S, HE_C)
    mo = masks[S * HE_C:].reshape(S, HE_C)

    wv2d = Wv.reshape(H, E, D, dh).transpose(2, 0, 1, 3).reshape(D, H * E * dh)
    v = pl.pallas_call(
        _v_body,
        grid=(S // VBLK,),
        in_specs=[
            pl.BlockSpec((VBLK, D), lambda i: (i, 0)),
            pl.BlockSpec((D, H * E * dh), lambda i: (0, 0)),
            pl.BlockSpec((VBLK, HE_C), lambda i: (i, 0)),
            pl.BlockSpec((VBLK, HE_C), lambda i: (i, 0)),
        ],
        out_specs=pl.BlockSpec((H, VBLK, DVA), lambda i: (0, i, 0)),
        out_shape=jax.ShapeDtypeStruct((H, S, DVA), _F32),
    )(x2d, wv2d, sel_v, mv)

    nab = S // ABLK
    tri = jnp.asarray(_TRI_NP)
    res = pl.pallas_call(
        _attn_body,
        grid=(H // HPB, nab, nab),
        in_specs=[
            pl.BlockSpec((HPB, ABLK, dh), lambda h, qi, kj: (h, qi, 0)),
            pl.BlockSpec((HPB, ABLK, dh),
                         lambda h, qi, kj: (h, jnp.minimum(kj, qi), 0)),
            pl.BlockSpec((HPB, ABLK, DVA),
                         lambda h, qi, kj: (h, jnp.minimum(kj, qi), 0)),
            pl.BlockSpec((1, ABLK, ABLK),
                         lambda h, qi, kj: (jnp.minimum(jnp.abs(qi - kj), 1),
                                            0, 0)),
        ],
        out_specs=pl.BlockSpec((HPB, ABLK, dh), lambda h, qi, kj: (h, qi, 0)),
        out_shape=jax.ShapeDtypeStruct((H, S, dh), _F32),
        scratch_shapes=[
            pltpu.VMEM((HPB, ABLK, DVA), _F32),
        ],
    )(q, k, v, tri)

    res2d = res.transpose(1, 0, 2).reshape(S, H * dh)
    wo2d = Wo.reshape(H, E, dh, D).transpose(1, 0, 2, 3).reshape(E * dh * H, D)
    gexp = jnp.asarray(_GEXP_NP)
    out2d = pl.pallas_call(
        _out_body,
        grid=(S // OBLK,),
        in_specs=[
            pl.BlockSpec((OBLK, H * dh), lambda i: (i, 0)),
            pl.BlockSpec((OBLK, HE_C), lambda i: (i, 0)),
            pl.BlockSpec((OBLK, HE_C), lambda i: (i, 0)),
            pl.BlockSpec((HE_C, E * D), lambda i: (0, 0)),
            pl.BlockSpec((E * dh * H, D), lambda i: (0, 0)),
        ],
        out_specs=pl.BlockSpec((OBLK, D), lambda i: (i, 0)),
        out_shape=jax.ShapeDtypeStruct((S, D), _F32),
    )(res2d, sel_o, mo, gexp, wo2d)

    return out2d.reshape(B, S, D)
